# baseline (speedup 1.0000x reference)
import functools

import jax
import jax.numpy as jnp
from jax import lax
from jax.experimental import pallas as pl
from jax.experimental.pallas import tpu as pltpu
from jax.experimental.pallas import tpu_sc as plsc

HEADS = 32
BUCKET_SIZE = 64
MAX_BUCKETS = 64
DIM = 256
TEMPERATURE = 0.7

BH = 64
BH_BLOCK = 4
LANES = 16
WORKERS = 32
BH_PER_WORKER = BH // WORKERS


def _tc_body(q_ref, k_ref, w_ref, o_ref):
    for b in range(BH_BLOCK):
        qs = jnp.sum(q_ref[b].reshape(MAX_BUCKETS, BUCKET_SIZE, 128), axis=1)
        ks = jnp.sum(k_ref[b].reshape(MAX_BUCKETS, BUCKET_SIZE, 128), axis=1)
        w = w_ref[0, b]
        r = jnp.dot(qs, w[:128, :], preferred_element_type=jnp.float32)
        r = r + jnp.dot(ks, w[128:, :], preferred_element_type=jnp.float32)
        r = jnp.maximum(r, 0.0)
        o_ref[b] = r.T


def _tc_logits(q, k, linear):
    return pl.pallas_call(
        _tc_body,
        grid=(BH // BH_BLOCK,),
        in_specs=[
            pl.BlockSpec((BH_BLOCK, 4096, 128), lambda i: (i, 0, 0)),
            pl.BlockSpec((BH_BLOCK, 4096, 128), lambda i: (i, 0, 0)),
            pl.BlockSpec((1, BH_BLOCK, DIM, MAX_BUCKETS),
                         lambda i: (0, i % (HEADS // BH_BLOCK), 0, 0)),
        ],
        out_specs=pl.BlockSpec((BH_BLOCK, MAX_BUCKETS, MAX_BUCKETS),
                               lambda i: (i, 0, 0)),
        out_shape=jax.ShapeDtypeStruct((BH, MAX_BUCKETS, MAX_BUCKETS),
                                       jnp.float32),
    )(q, k, linear)


PER_W = BH_PER_WORKER * MAX_BUCKETS * MAX_BUCKETS


def _sc_route_body(rt_hbm, out_hbm, rt_v, out_v):
    wid = lax.axis_index("s") * 2 + lax.axis_index("c")
    base = wid * PER_W
    pltpu.sync_copy(rt_hbm.at[pl.ds(base, PER_W)], rt_v)

    inv_t = jnp.float32(1.0 / TEMPERATURE)

    for b in range(BH_PER_WORKER):
        for rc in range(MAX_BUCKETS // LANES):
            off = b * MAX_BUCKETS * MAX_BUCKETS + rc * LANES

            def _scan_max(j, carry):
                m, idx = carry
                v = rt_v[pl.ds(off + j * MAX_BUCKETS, LANES)]
                gt = v > m
                return jnp.where(gt, v, m), jnp.where(gt, j, idx)

            m0 = jnp.full((LANES,), -jnp.inf, jnp.float32)
            i0 = jnp.zeros((LANES,), jnp.int32)
            m, idx = lax.fori_loop(0, MAX_BUCKETS, _scan_max, (m0, i0))

            def _scan_sum(j, s):
                v = rt_v[pl.ds(off + j * MAX_BUCKETS, LANES)]
                return s + jnp.exp((v - m) * inv_t)

            s = lax.fori_loop(0, MAX_BUCKETS, _scan_sum,
                              jnp.zeros((LANES,), jnp.float32))
            val = 1.0 / s

            def _write(j, _):
                out_v[pl.ds(off + j * MAX_BUCKETS, LANES)] = jnp.where(
                    idx == j, val, 0.0)
                return 0

            lax.fori_loop(0, MAX_BUCKETS, _write, 0)

    pltpu.sync_copy(out_v, out_hbm.at[pl.ds(base, PER_W)])


@functools.cache
def _sc_route():
    return pl.kernel(
        _sc_route_body,
        out_type=jax.ShapeDtypeStruct((BH * MAX_BUCKETS * MAX_BUCKETS,),
                                      jnp.float32),
        mesh=plsc.VectorSubcoreMesh(
            core_axis_name="c", subcore_axis_name="s",
            num_cores=2, num_subcores=16),
        scratch_types=[
            pltpu.VMEM((PER_W,), jnp.float32),
            pltpu.VMEM((PER_W,), jnp.float32),
        ],
    )


def _tr_body(i_ref, o_ref):
    for b in range(i_ref.shape[0]):
        o_ref[b] = i_ref[b].T


def _tc_untranspose(out_t):
    blk = 16
    return pl.pallas_call(
        _tr_body,
        grid=(BH // blk,),
        in_specs=[pl.BlockSpec((blk, MAX_BUCKETS, MAX_BUCKETS),
                               lambda i: (i, 0, 0))],
        out_specs=pl.BlockSpec((blk, MAX_BUCKETS, MAX_BUCKETS),
                               lambda i: (i, 0, 0)),
        out_shape=jax.ShapeDtypeStruct((BH, MAX_BUCKETS, MAX_BUCKETS),
                                       jnp.float32),
    )(out_t)


def kernel(q, k, linear, topk):
    rt = _tc_logits(q, k, linear)
    out_t = _sc_route()(rt.reshape(-1))
    return _tc_untranspose(out_t.reshape(BH, MAX_BUCKETS, MAX_BUCKETS))


# speedup vs baseline: 1.0274x; 1.0274x over previous
import functools

import jax
import jax.numpy as jnp
from jax import lax
from jax.experimental import pallas as pl
from jax.experimental.pallas import tpu as pltpu
from jax.experimental.pallas import tpu_sc as plsc

HEADS = 32
BUCKET_SIZE = 64
MAX_BUCKETS = 64
DIM = 256
TEMPERATURE = 0.7

BH = 64
BH_BLOCK = 4
LANES = 16
WORKERS = 32
BH_PER_WORKER = BH // WORKERS


def _tc_body(q_ref, k_ref, w_ref, o_ref):
    for b in range(BH_BLOCK):
        qs = jnp.sum(q_ref[b].reshape(MAX_BUCKETS, BUCKET_SIZE, 128), axis=1)
        ks = jnp.sum(k_ref[b].reshape(MAX_BUCKETS, BUCKET_SIZE, 128), axis=1)
        w = w_ref[0, b]
        r = jnp.dot(qs, w[:128, :], preferred_element_type=jnp.float32)
        r = r + jnp.dot(ks, w[128:, :], preferred_element_type=jnp.float32)
        r = jnp.maximum(r, 0.0)
        o_ref[b] = r.T


def _tc_logits(q, k, linear):
    return pl.pallas_call(
        _tc_body,
        grid=(BH // BH_BLOCK,),
        in_specs=[
            pl.BlockSpec((BH_BLOCK, 4096, 128), lambda i: (i, 0, 0)),
            pl.BlockSpec((BH_BLOCK, 4096, 128), lambda i: (i, 0, 0)),
            pl.BlockSpec((1, BH_BLOCK, DIM, MAX_BUCKETS),
                         lambda i: (0, i % (HEADS // BH_BLOCK), 0, 0)),
        ],
        out_specs=pl.BlockSpec((BH_BLOCK, MAX_BUCKETS, MAX_BUCKETS),
                               lambda i: (i, 0, 0)),
        out_shape=jax.ShapeDtypeStruct((BH, MAX_BUCKETS, MAX_BUCKETS),
                                       jnp.float32),
    )(q, k, linear)


PER_W = BH_PER_WORKER * MAX_BUCKETS * MAX_BUCKETS


def _sc_route_body(rt_hbm, out_hbm, rt_v, out_v):
    wid = lax.axis_index("s") * 2 + lax.axis_index("c")
    base = wid * PER_W
    pltpu.sync_copy(rt_hbm.at[pl.ds(base, PER_W)], rt_v)

    inv_t = jnp.float32(1.0 / TEMPERATURE)

    def _chunk(rc, _):
        off = (rc // 4) * MAX_BUCKETS * MAX_BUCKETS + (rc % 4) * LANES
        m = rt_v[pl.ds(off, LANES)]
        idx = jnp.zeros((LANES,), jnp.int32)
        for j in range(1, MAX_BUCKETS):
            v = rt_v[pl.ds(off + j * MAX_BUCKETS, LANES)]
            gt = v > m
            m = jnp.where(gt, v, m)
            idx = jnp.where(gt, j, idx)
        s = jnp.zeros((LANES,), jnp.float32)
        for j in range(MAX_BUCKETS):
            v = rt_v[pl.ds(off + j * MAX_BUCKETS, LANES)]
            s = s + jnp.exp((v - m) * inv_t)
        val = 1.0 / s
        for j in range(MAX_BUCKETS):
            out_v[pl.ds(off + j * MAX_BUCKETS, LANES)] = jnp.where(
                idx == j, val, 0.0)
        return 0

    lax.fori_loop(0, 2 * 4, _chunk, 0)

    pltpu.sync_copy(out_v, out_hbm.at[pl.ds(base, PER_W)])


@functools.cache
def _sc_route():
    return pl.kernel(
        _sc_route_body,
        out_type=jax.ShapeDtypeStruct((BH * MAX_BUCKETS * MAX_BUCKETS,),
                                      jnp.float32),
        mesh=plsc.VectorSubcoreMesh(
            core_axis_name="c", subcore_axis_name="s",
            num_cores=2, num_subcores=16),
        scratch_types=[
            pltpu.VMEM((PER_W,), jnp.float32),
            pltpu.VMEM((PER_W,), jnp.float32),
        ],
    )


def _tr_body(i_ref, o_ref):
    for b in range(i_ref.shape[0]):
        o_ref[b] = i_ref[b].T


def _tc_untranspose(out_t):
    blk = 16
    return pl.pallas_call(
        _tr_body,
        grid=(BH // blk,),
        in_specs=[pl.BlockSpec((blk, MAX_BUCKETS, MAX_BUCKETS),
                               lambda i: (i, 0, 0))],
        out_specs=pl.BlockSpec((blk, MAX_BUCKETS, MAX_BUCKETS),
                               lambda i: (i, 0, 0)),
        out_shape=jax.ShapeDtypeStruct((BH, MAX_BUCKETS, MAX_BUCKETS),
                                       jnp.float32),
    )(out_t)


def kernel(q, k, linear, topk):
    rt = _tc_logits(q, k, linear)
    out_t = _sc_route()(rt.reshape(-1))
    return _tc_untranspose(out_t.reshape(BH, MAX_BUCKETS, MAX_BUCKETS))
